# baseline (speedup 1.0000x reference)
import functools

import jax
import jax.numpy as jnp
from jax import lax
from jax.experimental import pallas as pl
from jax.experimental.pallas import tpu as pltpu
from jax.experimental.pallas import tpu_sc as plsc

N = 10000
E = 320000
D = 128
H = 128
C = 10
G = 64

NC = 2
NS = 16
NW = NC * NS

K = 32
EW = 10240
CH = EW // K
NG = CH // 8
NP = 10240
ROWS = NP // NS
SC_CH = K
OC = ROWS // SC_CH


def _sc_agg_body(table, src3d, dst3d, zeros_h, acc_out,
                 src_g, dst_g, rows0, rows1, rows2, rows3, rows4, rows5,
                 rows6, rows7, acc_sh, semg0, semg1, semg2, semg3, semg4,
                 semg5, semg6, semg7, semi1, semi2):
    c = lax.axis_index("c")
    s = lax.axis_index("s")
    w = s * NC + c
    rows = (rows0, rows1, rows2, rows3, rows4, rows5, rows6, rows7)
    semg = (semg0, semg1, semg2, semg3, semg4, semg5, semg6, semg7)

    pltpu.sync_copy(zeros_h.at[pl.ds(s * ROWS, ROWS)],
                    acc_sh.at[pl.ds(s * ROWS, ROWS)])
    plsc.subcore_barrier()

    pltpu.sync_copy(src3d.at[w, pl.ds(0, 8)], src_g.at[pl.ds(0, 8)])
    pltpu.sync_copy(dst3d.at[w, pl.ds(0, 8)], dst_g.at[pl.ds(0, 8)])
    for b in range(6):
        pltpu.async_copy(table.at[src_g.at[b]], rows[b], semg[b])

    def group(gr, carry):
        cur = lax.rem(gr, 2)
        nxt = lax.rem(gr + 1, 2)
        gf = jnp.minimum(gr + 1, NG - 1)
        gof = pl.multiple_of(gf * 8, 8)
        di1 = pltpu.async_copy(src3d.at[w, pl.ds(gof, 8)],
                               src_g.at[pl.ds(nxt * 8, 8)], semi1)
        di2 = pltpu.async_copy(dst3d.at[w, pl.ds(gof, 8)],
                               dst_g.at[pl.ds(nxt * 8, 8)], semi2)
        dgs = [None, None]
        for r in range(8):
            jrow = cur * 8 + r
            p = r % 8
            if r < 6:
                pltpu.make_async_copy(table.at[src_g.at[jrow]], rows[p],
                                      semg[p]).wait()
            else:
                dgs[r - 6].wait()
            if r == 2:
                di1.wait()
                di2.wait()
            nrow = jrow + 6 if r < 2 else nxt * 8 + (r - 2)
            pn = (r + 6) % 8
            dg = pltpu.async_copy(table.at[src_g.at[nrow]],
                                  rows[pn], semg[pn])
            if r >= 6:
                pass
            if r < 2:
                dgs[r] = dg
            pltpu.sync_copy(rows[p], acc_sh.at[dst_g.at[jrow]], add=True)
        return carry

    lax.fori_loop(0, NG, group, 0)
    for b in range(6):
        pltpu.make_async_copy(table.at[src_g.at[b]], rows[b], semg[b]).wait()
    plsc.subcore_barrier()

    pltpu.sync_copy(acc_sh.at[pl.ds(s * ROWS, ROWS)],
                    acc_out.at[c, pl.ds(s * ROWS, ROWS)])


def _sc_count_body(dst3d, zeros_h, ones_h, cnt_out,
                   dst_g, ones_v, cnt_sh, semi2, sems0):
    c = lax.axis_index("c")
    s = lax.axis_index("s")
    w = s * NC + c

    pltpu.sync_copy(zeros_h.at[pl.ds(s * ROWS, ROWS)],
                    cnt_sh.at[pl.ds(s * ROWS, ROWS)])
    pltpu.sync_copy(ones_h, ones_v)
    plsc.subcore_barrier()
    pltpu.sync_copy(dst3d.at[w, pl.ds(0, 8)], dst_g.at[pl.ds(0, 8)])

    def group(gr, carry):
        cur = lax.rem(gr, 2)
        nxt = lax.rem(gr + 1, 2)
        gf = jnp.minimum(gr + 1, NG - 1)
        gof = pl.multiple_of(gf * 8, 8)
        di2 = pltpu.async_copy(dst3d.at[w, pl.ds(gof, 8)],
                               dst_g.at[pl.ds(nxt * 8, 8)], semi2)
        dss = []
        for r in range(8):
            dss.append(pltpu.async_copy(
                ones_v, cnt_sh.at[dst_g.at[cur * 8 + r]], sems0, add=True))
        for dsd in dss:
            dsd.wait()
        di2.wait()
        return carry

    lax.fori_loop(0, NG, group, 0)
    plsc.subcore_barrier()

    pltpu.sync_copy(cnt_sh.at[pl.ds(s * ROWS, ROWS)],
                    cnt_out.at[c, pl.ds(s * ROWS, ROWS)])


def _sc_mesh():
    return plsc.VectorSubcoreMesh(core_axis_name="c", subcore_axis_name="s",
                                  num_cores=NC, num_subcores=NS)


@functools.lru_cache(maxsize=None)
def _make_sc_agg():
    scratch = [
        pltpu.VMEM((2 * 8, K), jnp.int32),
        pltpu.VMEM((2 * 8, K), jnp.int32),
    ] + [pltpu.VMEM((K, D), jnp.float32)] * 8 + [
        pltpu.VMEM_SHARED((NP, D), jnp.float32),
    ] + [pltpu.SemaphoreType.DMA] * 10
    return pl.kernel(_sc_agg_body,
                     out_type=[jax.ShapeDtypeStruct((NC, NP, D), jnp.float32)],
                     mesh=_sc_mesh(), scratch_types=scratch, name="sc_agg")


@functools.lru_cache(maxsize=None)
def _make_sc_count():
    scratch = [
        pltpu.VMEM((2 * 8, K), jnp.int32),
        pltpu.VMEM((K, D), jnp.float32),
        pltpu.VMEM_SHARED((NP, D), jnp.float32),
        pltpu.SemaphoreType.DMA,
        pltpu.SemaphoreType.DMA,
    ]
    return pl.kernel(_sc_count_body,
                     out_type=[jax.ShapeDtypeStruct((NC, NP, D), jnp.float32)],
                     mesh=_sc_mesh(), scratch_types=scratch, name="sc_count")


_BN = 1000
_NB = N // _BN


def _tc_layer_common(acc_ref, cnt_ref, h_ref, b_ref, wl_ref, bl_ref, wr_ref,
                     ho_ref, pool_ref):
    i = pl.program_id(0)
    cnt = cnt_ref[0, :, 0:1] + cnt_ref[1, :, 0:1]
    inv = 1.0 / jnp.maximum(cnt, 1.0)
    agg = (acc_ref[0] + acc_ref[1]) * inv
    hmat = h_ref[...]
    out = (lax.dot_general(agg, wl_ref[...], (((1,), (1,)), ((), ())),
                           preferred_element_type=jnp.float32)
           + bl_ref[...]
           + lax.dot_general(hmat, wr_ref[...], (((1,), (1,)), ((), ())),
                             preferred_element_type=jnp.float32))
    ho_ref[...] = out
    onehot_t = (lax.broadcasted_iota(jnp.int32, (G, _BN), 0)
                == b_ref[0]).astype(jnp.float32)
    p = lax.dot_general(onehot_t, out, (((1,), (0,)), ((), ())),
                        preferred_element_type=jnp.float32)

    @pl.when(i == 0)
    def _():
        pool_ref[...] = jnp.zeros_like(pool_ref)

    pool_ref[...] += p


def _tc_layer_body(acc_ref, cnt_ref, h_ref, b_ref, wl_ref, bl_ref, wr_ref,
                   ho_ref, pool_ref):
    _tc_layer_common(acc_ref, cnt_ref, h_ref, b_ref, wl_ref, bl_ref, wr_ref,
                     ho_ref, pool_ref)


def _tc_layer3_body(acc_ref, cnt_ref, h_ref, b_ref, wl_ref, bl_ref, wr_ref,
                    p1_ref, p2_ref, w1_ref, b1_ref, w2_ref, b2_ref,
                    ho_ref, pool_ref, o_ref):
    _tc_layer_common(acc_ref, cnt_ref, h_ref, b_ref, wl_ref, bl_ref, wr_ref,
                     ho_ref, pool_ref)

    @pl.when(pl.program_id(0) == _NB - 1)
    def _():
        dn = (((1,), (1,)), ((), ()))
        z = (lax.dot_general(p1_ref[...], w1_ref[:, 0:H], dn,
                             preferred_element_type=jnp.float32)
             + lax.dot_general(p2_ref[...], w1_ref[:, H:2 * H], dn,
                               preferred_element_type=jnp.float32)
             + lax.dot_general(pool_ref[...], w1_ref[:, 2 * H:3 * H], dn,
                               preferred_element_type=jnp.float32))
        z = jnp.maximum(z + b1_ref[...], 0.0)
        z2 = lax.dot_general(z, w2_ref[...], dn,
                             preferred_element_type=jnp.float32) + b2_ref[...]
        m = jnp.max(z2, axis=-1, keepdims=True)
        lse = jnp.log(jnp.sum(jnp.exp(z2 - m), axis=-1, keepdims=True)) + m
        o_ref[...] = z2 - lse


_tc_layer = pl.pallas_call(
    _tc_layer_body,
    grid=(_NB,),
    in_specs=[
        pl.BlockSpec((NC, _BN, D), lambda i: (0, i, 0)),
        pl.BlockSpec((NC, _BN, D), lambda i: (0, i, 0)),
        pl.BlockSpec((_BN, D), lambda i: (i, 0)),
        pl.BlockSpec((1, 1, _BN), lambda i: (i, 0, 0)),
        pl.BlockSpec((H, D), lambda i: (0, 0)),
        pl.BlockSpec((1, H), lambda i: (0, 0)),
        pl.BlockSpec((H, D), lambda i: (0, 0)),
    ],
    out_specs=[
        pl.BlockSpec((_BN, H), lambda i: (i, 0)),
        pl.BlockSpec((G, H), lambda i: (0, 0)),
    ],
    out_shape=[
        jax.ShapeDtypeStruct((N, H), jnp.float32),
        jax.ShapeDtypeStruct((G, H), jnp.float32),
    ],
)


_tc_layer3 = pl.pallas_call(
    _tc_layer3_body,
    grid=(_NB,),
    in_specs=[
        pl.BlockSpec((NC, _BN, D), lambda i: (0, i, 0)),
        pl.BlockSpec((NC, _BN, D), lambda i: (0, i, 0)),
        pl.BlockSpec((_BN, D), lambda i: (i, 0)),
        pl.BlockSpec((1, 1, _BN), lambda i: (i, 0, 0)),
        pl.BlockSpec((H, D), lambda i: (0, 0)),
        pl.BlockSpec((1, H), lambda i: (0, 0)),
        pl.BlockSpec((H, D), lambda i: (0, 0)),
        pl.BlockSpec((G, H), lambda i: (0, 0)),
        pl.BlockSpec((G, H), lambda i: (0, 0)),
        pl.BlockSpec((H, 3 * H), lambda i: (0, 0)),
        pl.BlockSpec((1, H), lambda i: (0, 0)),
        pl.BlockSpec((C, H), lambda i: (0, 0)),
        pl.BlockSpec((1, C), lambda i: (0, 0)),
    ],
    out_specs=[
        pl.BlockSpec((_BN, H), lambda i: (i, 0)),
        pl.BlockSpec((G, H), lambda i: (0, 0)),
        pl.BlockSpec((G, C), lambda i: (0, 0)),
    ],
    out_shape=[
        jax.ShapeDtypeStruct((N, H), jnp.float32),
        jax.ShapeDtypeStruct((G, H), jnp.float32),
        jax.ShapeDtypeStruct((G, C), jnp.float32),
    ],
)


def _tc_head_body(p1_ref, p2_ref, p3_ref, w1_ref, b1_ref, w2_ref, b2_ref,
                  o_ref):
    dn = (((1,), (1,)), ((), ()))
    z = (lax.dot_general(p1_ref[...], w1_ref[:, 0:H], dn,
                         preferred_element_type=jnp.float32)
         + lax.dot_general(p2_ref[...], w1_ref[:, H:2 * H], dn,
                           preferred_element_type=jnp.float32)
         + lax.dot_general(p3_ref[...], w1_ref[:, 2 * H:3 * H], dn,
                           preferred_element_type=jnp.float32))
    z = jnp.maximum(z + b1_ref[...], 0.0)
    z2 = lax.dot_general(z, w2_ref[...], dn,
                         preferred_element_type=jnp.float32) + b2_ref[...]
    m = jnp.max(z2, axis=-1, keepdims=True)
    lse = jnp.log(jnp.sum(jnp.exp(z2 - m), axis=-1, keepdims=True)) + m
    o_ref[...] = z2 - lse


_tc_head = pl.pallas_call(
    _tc_head_body,
    out_shape=jax.ShapeDtypeStruct((G, C), jnp.float32),
)


def kernel(x, edge_index, batch, Wl0, bl0, Wr0, Wl1, bl1, Wr1, Wl2, bl2,
           Wr2, W1, b1, W2, b2):
    npad = EW - E // NW
    pad_src = (jnp.arange(NW * npad, dtype=jnp.int32) % 64).reshape(NW, npad)
    pad_dst = (N + jnp.arange(NW * npad, dtype=jnp.int32)
               % (NP - N)).reshape(NW, npad)
    src3d = jnp.concatenate(
        [edge_index[0].reshape(NW, E // NW), pad_src], 1).reshape(NW, CH, K)
    dst3d = jnp.concatenate(
        [edge_index[1].reshape(NW, E // NW), pad_dst], 1).reshape(NW, CH, K)
    zeros_h = jnp.zeros((NP, D), jnp.float32)
    ones_h = jnp.ones((K, D), jnp.float32)
    batch3d = batch.reshape(_NB, 1, _BN)

    cntp, = _make_sc_count()(dst3d, zeros_h, ones_h)
    acc, = _make_sc_agg()(x, src3d, dst3d, zeros_h)
    h1, p1 = _tc_layer(acc, cntp, x, batch3d, Wl0, bl0.reshape(1, H), Wr0)
    acc, = _make_sc_agg()(h1, src3d, dst3d, zeros_h)
    h2, p2 = _tc_layer(acc, cntp, h1, batch3d, Wl1, bl1.reshape(1, H), Wr1)
    acc, = _make_sc_agg()(h2, src3d, dst3d, zeros_h)
    _, _, out = _tc_layer3(acc, cntp, h2, batch3d, Wl2, bl2.reshape(1, H),
                           Wr2, p1, p2, W1, b1.reshape(1, H), W2,
                           b2.reshape(1, C))
    return out


# speedup vs baseline: 1.0896x; 1.0896x over previous
import functools

import jax
import jax.numpy as jnp
from jax import lax
from jax.experimental import pallas as pl
from jax.experimental.pallas import tpu as pltpu
from jax.experimental.pallas import tpu_sc as plsc

N = 10000
E = 320000
D = 128
H = 128
C = 10
G = 64

NC = 2
NS = 16
NW = NC * NS

K = 64
EW = 10240
CH = EW // K
NG = CH // 8
NP = 10240
ROWS = NP // NS
SC_CH = K
OC = ROWS // SC_CH


def _sc_agg_body(table, src3d, dst3d, zeros_h, acc_out,
                 src_g, dst_g, rows0, rows1, rows2, rows3, acc_sh,
                 semg0, semg1, semg2, semg3, semi1, semi2):
    c = lax.axis_index("c")
    s = lax.axis_index("s")
    w = s * NC + c
    rows = (rows0, rows1, rows2, rows3)
    semg = (semg0, semg1, semg2, semg3)

    pltpu.sync_copy(zeros_h.at[pl.ds(s * ROWS, ROWS)],
                    acc_sh.at[pl.ds(s * ROWS, ROWS)])
    plsc.subcore_barrier()

    pltpu.sync_copy(src3d.at[w, pl.ds(0, 8)], src_g.at[pl.ds(0, 8)])
    pltpu.sync_copy(dst3d.at[w, pl.ds(0, 8)], dst_g.at[pl.ds(0, 8)])
    pltpu.async_copy(table.at[src_g.at[0]], rows0, semg0)
    pltpu.async_copy(table.at[src_g.at[1]], rows1, semg1)
    pltpu.async_copy(table.at[src_g.at[2]], rows2, semg2)

    def group(gr, carry):
        cur = lax.rem(gr, 2)
        nxt = lax.rem(gr + 1, 2)
        gf = jnp.minimum(gr + 1, NG - 1)
        gof = pl.multiple_of(gf * 8, 8)
        di1 = pltpu.async_copy(src3d.at[w, pl.ds(gof, 8)],
                               src_g.at[pl.ds(nxt * 8, 8)], semi1)
        di2 = pltpu.async_copy(dst3d.at[w, pl.ds(gof, 8)],
                               dst_g.at[pl.ds(nxt * 8, 8)], semi2)
        dgs = [None, None, None]
        for r in range(8):
            jrow = cur * 8 + r
            p = r % 4
            if r < 3:
                pltpu.make_async_copy(table.at[src_g.at[jrow]], rows[p],
                                      semg[p]).wait()
            else:
                dgs[r % 3].wait()
            if r == 5:
                di1.wait()
                di2.wait()
            nrow = jrow + 3 if r < 5 else nxt * 8 + (r - 5)
            pn = (r + 3) % 4
            dgs[r % 3] = pltpu.async_copy(table.at[src_g.at[nrow]],
                                          rows[pn], semg[pn])
            pltpu.sync_copy(rows[p], acc_sh.at[dst_g.at[jrow]], add=True)
        return carry

    lax.fori_loop(0, NG, group, 0)
    pltpu.make_async_copy(table.at[src_g.at[0]], rows0, semg0).wait()
    pltpu.make_async_copy(table.at[src_g.at[1]], rows1, semg1).wait()
    pltpu.make_async_copy(table.at[src_g.at[2]], rows2, semg2).wait()
    plsc.subcore_barrier()

    pltpu.sync_copy(acc_sh.at[pl.ds(s * ROWS, ROWS)],
                    acc_out.at[c, pl.ds(s * ROWS, ROWS)])


def _sc_count_body(dst3d, zeros_h, ones_h, cnt_out,
                   dst_g, ones_v, cnt_sh, semi2, sems0):
    c = lax.axis_index("c")
    s = lax.axis_index("s")
    w = s * NC + c

    pltpu.sync_copy(zeros_h.at[pl.ds(s * ROWS, ROWS)],
                    cnt_sh.at[pl.ds(s * ROWS, ROWS)])
    pltpu.sync_copy(ones_h, ones_v)
    plsc.subcore_barrier()
    pltpu.sync_copy(dst3d.at[w, pl.ds(0, 8)], dst_g.at[pl.ds(0, 8)])

    def group(gr, carry):
        cur = lax.rem(gr, 2)
        nxt = lax.rem(gr + 1, 2)
        gf = jnp.minimum(gr + 1, NG - 1)
        gof = pl.multiple_of(gf * 8, 8)
        di2 = pltpu.async_copy(dst3d.at[w, pl.ds(gof, 8)],
                               dst_g.at[pl.ds(nxt * 8, 8)], semi2)
        dss = []
        for r in range(8):
            dss.append(pltpu.async_copy(
                ones_v, cnt_sh.at[dst_g.at[cur * 8 + r]], sems0, add=True))
        for dsd in dss:
            dsd.wait()
        di2.wait()
        return carry

    lax.fori_loop(0, NG, group, 0)
    plsc.subcore_barrier()

    pltpu.sync_copy(cnt_sh.at[pl.ds(s * ROWS, ROWS)],
                    cnt_out.at[c, pl.ds(s * ROWS, ROWS)])


def _sc_mesh():
    return plsc.VectorSubcoreMesh(core_axis_name="c", subcore_axis_name="s",
                                  num_cores=NC, num_subcores=NS)


@functools.lru_cache(maxsize=None)
def _make_sc_agg():
    scratch = [
        pltpu.VMEM((2 * 8, K), jnp.int32),
        pltpu.VMEM((2 * 8, K), jnp.int32),
    ] + [pltpu.VMEM((K, D), jnp.float32)] * 4 + [
        pltpu.VMEM_SHARED((NP, D), jnp.float32),
    ] + [pltpu.SemaphoreType.DMA] * 6
    return pl.kernel(_sc_agg_body,
                     out_type=[jax.ShapeDtypeStruct((NC, NP, D), jnp.float32)],
                     mesh=_sc_mesh(), scratch_types=scratch, name="sc_agg")


@functools.lru_cache(maxsize=None)
def _make_sc_count():
    scratch = [
        pltpu.VMEM((2 * 8, K), jnp.int32),
        pltpu.VMEM((K, D), jnp.float32),
        pltpu.VMEM_SHARED((NP, D), jnp.float32),
        pltpu.SemaphoreType.DMA,
        pltpu.SemaphoreType.DMA,
    ]
    return pl.kernel(_sc_count_body,
                     out_type=[jax.ShapeDtypeStruct((NC, NP, D), jnp.float32)],
                     mesh=_sc_mesh(), scratch_types=scratch, name="sc_count")


_BN = 1000
_NB = N // _BN


def _tc_layer_common(acc_ref, cnt_ref, h_ref, b_ref, wl_ref, bl_ref, wr_ref,
                     ho_ref, pool_ref):
    i = pl.program_id(0)
    cnt = cnt_ref[0, :, 0:1] + cnt_ref[1, :, 0:1]
    inv = 1.0 / jnp.maximum(cnt, 1.0)
    agg = (acc_ref[0] + acc_ref[1]) * inv
    hmat = h_ref[...]
    out = (lax.dot_general(agg, wl_ref[...], (((1,), (1,)), ((), ())),
                           preferred_element_type=jnp.float32)
           + bl_ref[...]
           + lax.dot_general(hmat, wr_ref[...], (((1,), (1,)), ((), ())),
                             preferred_element_type=jnp.float32))
    ho_ref[...] = out
    onehot_t = (lax.broadcasted_iota(jnp.int32, (G, _BN), 0)
                == b_ref[0]).astype(jnp.float32)
    p = lax.dot_general(onehot_t, out, (((1,), (0,)), ((), ())),
                        preferred_element_type=jnp.float32)

    @pl.when(i == 0)
    def _():
        pool_ref[...] = jnp.zeros_like(pool_ref)

    pool_ref[...] += p


def _tc_layer_body(acc_ref, cnt_ref, h_ref, b_ref, wl_ref, bl_ref, wr_ref,
                   ho_ref, pool_ref):
    _tc_layer_common(acc_ref, cnt_ref, h_ref, b_ref, wl_ref, bl_ref, wr_ref,
                     ho_ref, pool_ref)


def _tc_layer3_body(acc_ref, cnt_ref, h_ref, b_ref, wl_ref, bl_ref, wr_ref,
                    p1_ref, p2_ref, w1_ref, b1_ref, w2_ref, b2_ref,
                    ho_ref, pool_ref, o_ref):
    _tc_layer_common(acc_ref, cnt_ref, h_ref, b_ref, wl_ref, bl_ref, wr_ref,
                     ho_ref, pool_ref)

    @pl.when(pl.program_id(0) == _NB - 1)
    def _():
        dn = (((1,), (1,)), ((), ()))
        z = (lax.dot_general(p1_ref[...], w1_ref[:, 0:H], dn,
                             preferred_element_type=jnp.float32)
             + lax.dot_general(p2_ref[...], w1_ref[:, H:2 * H], dn,
                               preferred_element_type=jnp.float32)
             + lax.dot_general(pool_ref[...], w1_ref[:, 2 * H:3 * H], dn,
                               preferred_element_type=jnp.float32))
        z = jnp.maximum(z + b1_ref[...], 0.0)
        z2 = lax.dot_general(z, w2_ref[...], dn,
                             preferred_element_type=jnp.float32) + b2_ref[...]
        m = jnp.max(z2, axis=-1, keepdims=True)
        lse = jnp.log(jnp.sum(jnp.exp(z2 - m), axis=-1, keepdims=True)) + m
        o_ref[...] = z2 - lse


_tc_layer = pl.pallas_call(
    _tc_layer_body,
    grid=(_NB,),
    in_specs=[
        pl.BlockSpec((NC, _BN, D), lambda i: (0, i, 0)),
        pl.BlockSpec((NC, _BN, D), lambda i: (0, i, 0)),
        pl.BlockSpec((_BN, D), lambda i: (i, 0)),
        pl.BlockSpec((1, 1, _BN), lambda i: (i, 0, 0)),
        pl.BlockSpec((H, D), lambda i: (0, 0)),
        pl.BlockSpec((1, H), lambda i: (0, 0)),
        pl.BlockSpec((H, D), lambda i: (0, 0)),
    ],
    out_specs=[
        pl.BlockSpec((_BN, H), lambda i: (i, 0)),
        pl.BlockSpec((G, H), lambda i: (0, 0)),
    ],
    out_shape=[
        jax.ShapeDtypeStruct((N, H), jnp.float32),
        jax.ShapeDtypeStruct((G, H), jnp.float32),
    ],
)


_tc_layer3 = pl.pallas_call(
    _tc_layer3_body,
    grid=(_NB,),
    in_specs=[
        pl.BlockSpec((NC, _BN, D), lambda i: (0, i, 0)),
        pl.BlockSpec((NC, _BN, D), lambda i: (0, i, 0)),
        pl.BlockSpec((_BN, D), lambda i: (i, 0)),
        pl.BlockSpec((1, 1, _BN), lambda i: (i, 0, 0)),
        pl.BlockSpec((H, D), lambda i: (0, 0)),
        pl.BlockSpec((1, H), lambda i: (0, 0)),
        pl.BlockSpec((H, D), lambda i: (0, 0)),
        pl.BlockSpec((G, H), lambda i: (0, 0)),
        pl.BlockSpec((G, H), lambda i: (0, 0)),
        pl.BlockSpec((H, 3 * H), lambda i: (0, 0)),
        pl.BlockSpec((1, H), lambda i: (0, 0)),
        pl.BlockSpec((C, H), lambda i: (0, 0)),
        pl.BlockSpec((1, C), lambda i: (0, 0)),
    ],
    out_specs=[
        pl.BlockSpec((_BN, H), lambda i: (i, 0)),
        pl.BlockSpec((G, H), lambda i: (0, 0)),
        pl.BlockSpec((G, C), lambda i: (0, 0)),
    ],
    out_shape=[
        jax.ShapeDtypeStruct((N, H), jnp.float32),
        jax.ShapeDtypeStruct((G, H), jnp.float32),
        jax.ShapeDtypeStruct((G, C), jnp.float32),
    ],
)


def _tc_head_body(p1_ref, p2_ref, p3_ref, w1_ref, b1_ref, w2_ref, b2_ref,
                  o_ref):
    dn = (((1,), (1,)), ((), ()))
    z = (lax.dot_general(p1_ref[...], w1_ref[:, 0:H], dn,
                         preferred_element_type=jnp.float32)
         + lax.dot_general(p2_ref[...], w1_ref[:, H:2 * H], dn,
                           preferred_element_type=jnp.float32)
         + lax.dot_general(p3_ref[...], w1_ref[:, 2 * H:3 * H], dn,
                           preferred_element_type=jnp.float32))
    z = jnp.maximum(z + b1_ref[...], 0.0)
    z2 = lax.dot_general(z, w2_ref[...], dn,
                         preferred_element_type=jnp.float32) + b2_ref[...]
    m = jnp.max(z2, axis=-1, keepdims=True)
    lse = jnp.log(jnp.sum(jnp.exp(z2 - m), axis=-1, keepdims=True)) + m
    o_ref[...] = z2 - lse


_tc_head = pl.pallas_call(
    _tc_head_body,
    out_shape=jax.ShapeDtypeStruct((G, C), jnp.float32),
)


def kernel(x, edge_index, batch, Wl0, bl0, Wr0, Wl1, bl1, Wr1, Wl2, bl2,
           Wr2, W1, b1, W2, b2):
    npad = EW - E // NW
    pad_src = (jnp.arange(NW * npad, dtype=jnp.int32) % 64).reshape(NW, npad)
    pad_dst = (N + jnp.arange(NW * npad, dtype=jnp.int32)
               % (NP - N)).reshape(NW, npad)
    src3d = jnp.concatenate(
        [edge_index[0].reshape(NW, E // NW), pad_src], 1).reshape(NW, CH, K)
    dst3d = jnp.concatenate(
        [edge_index[1].reshape(NW, E // NW), pad_dst], 1).reshape(NW, CH, K)
    zeros_h = jnp.zeros((NP, D), jnp.float32)
    ones_h = jnp.ones((K, D), jnp.float32)
    batch3d = batch.reshape(_NB, 1, _BN)

    cntp, = _make_sc_count()(dst3d, zeros_h, ones_h)
    acc, = _make_sc_agg()(x, src3d, dst3d, zeros_h)
    h1, p1 = _tc_layer(acc, cntp, x, batch3d, Wl0, bl0.reshape(1, H), Wr0)
    acc, = _make_sc_agg()(h1, src3d, dst3d, zeros_h)
    h2, p2 = _tc_layer(acc, cntp, h1, batch3d, Wl1, bl1.reshape(1, H), Wr1)
    acc, = _make_sc_agg()(h2, src3d, dst3d, zeros_h)
    _, _, out = _tc_layer3(acc, cntp, h2, batch3d, Wl2, bl2.reshape(1, H),
                           Wr2, p1, p2, W1, b1.reshape(1, H), W2,
                           b2.reshape(1, C))
    return out


# speedup vs baseline: 1.1173x; 1.0254x over previous
import functools

import jax
import jax.numpy as jnp
from jax import lax
from jax.experimental import pallas as pl
from jax.experimental.pallas import tpu as pltpu
from jax.experimental.pallas import tpu_sc as plsc

N = 10000
E = 320000
D = 128
H = 128
C = 10
G = 64

NC = 2
NS = 16
NW = NC * NS

K = 64
EW = 10240
CH = EW // K
NG = CH // 8
K2 = 128
CH2 = EW // K2
NG2 = CH2 // 8
NP = 10240
ROWS = NP // NS
SC_CH = K
OC = ROWS // SC_CH


def _sc_agg_body(table, src3d, dst3d, zeros_h, acc_out,
                 src_g, dst_g, rows0, rows1, rows2, rows3, acc_sh,
                 semg0, semg1, semg2, semg3, semi1, semi2):
    c = lax.axis_index("c")
    s = lax.axis_index("s")
    w = s * NC + c
    rows = (rows0, rows1, rows2, rows3)
    semg = (semg0, semg1, semg2, semg3)

    pltpu.sync_copy(zeros_h.at[pl.ds(s * ROWS, ROWS)],
                    acc_sh.at[pl.ds(s * ROWS, ROWS)])
    plsc.subcore_barrier()

    pltpu.sync_copy(src3d.at[w, pl.ds(0, 8)], src_g.at[pl.ds(0, 8)])
    pltpu.sync_copy(dst3d.at[w, pl.ds(0, 8)], dst_g.at[pl.ds(0, 8)])
    pltpu.async_copy(table.at[src_g.at[0]], rows0, semg0)
    pltpu.async_copy(table.at[src_g.at[1]], rows1, semg1)
    pltpu.async_copy(table.at[src_g.at[2]], rows2, semg2)

    def group(gr, carry):
        cur = lax.rem(gr, 2)
        nxt = lax.rem(gr + 1, 2)
        gf = jnp.minimum(gr + 1, NG - 1)
        gof = pl.multiple_of(gf * 8, 8)
        di1 = pltpu.async_copy(src3d.at[w, pl.ds(gof, 8)],
                               src_g.at[pl.ds(nxt * 8, 8)], semi1)
        di2 = pltpu.async_copy(dst3d.at[w, pl.ds(gof, 8)],
                               dst_g.at[pl.ds(nxt * 8, 8)], semi2)
        dgs = [None, None, None]
        for r in range(8):
            jrow = cur * 8 + r
            p = r % 4
            if r < 3:
                pltpu.make_async_copy(table.at[src_g.at[jrow]], rows[p],
                                      semg[p]).wait()
            else:
                dgs[r % 3].wait()
            if r == 5:
                di1.wait()
                di2.wait()
            nrow = jrow + 3 if r < 5 else nxt * 8 + (r - 5)
            pn = (r + 3) % 4
            dgs[r % 3] = pltpu.async_copy(table.at[src_g.at[nrow]],
                                          rows[pn], semg[pn])
            pltpu.sync_copy(rows[p], acc_sh.at[dst_g.at[jrow]], add=True)
        return carry

    lax.fori_loop(0, NG, group, 0)
    pltpu.make_async_copy(table.at[src_g.at[0]], rows0, semg0).wait()
    pltpu.make_async_copy(table.at[src_g.at[1]], rows1, semg1).wait()
    pltpu.make_async_copy(table.at[src_g.at[2]], rows2, semg2).wait()
    plsc.subcore_barrier()

    pltpu.sync_copy(acc_sh.at[pl.ds(s * ROWS, ROWS)],
                    acc_out.at[c, pl.ds(s * ROWS, ROWS)])


def _sc_count_body(dst3d2, zeros_h, ones_h, cnt_out,
                   dst_g, ones_v, cnt_sh, semi2, sems0):
    c = lax.axis_index("c")
    s = lax.axis_index("s")
    w = s * NC + c

    pltpu.sync_copy(zeros_h.at[pl.ds(s * ROWS, ROWS)],
                    cnt_sh.at[pl.ds(s * ROWS, ROWS)])
    pltpu.sync_copy(ones_h, ones_v)
    plsc.subcore_barrier()
    pltpu.sync_copy(dst3d2.at[w, pl.ds(0, 8)], dst_g.at[pl.ds(0, 8)])

    def group(gr, carry):
        cur = lax.rem(gr, 2)
        nxt = lax.rem(gr + 1, 2)
        gf = jnp.minimum(gr + 1, NG2 - 1)
        gof = pl.multiple_of(gf * 8, 8)
        di2 = pltpu.async_copy(dst3d2.at[w, pl.ds(gof, 8)],
                               dst_g.at[pl.ds(nxt * 8, 8)], semi2)
        dss = []
        for r in range(8):
            dss.append(pltpu.async_copy(
                ones_v, cnt_sh.at[dst_g.at[cur * 8 + r]], sems0, add=True))
        for dsd in dss:
            dsd.wait()
        di2.wait()
        return carry

    lax.fori_loop(0, NG2, group, 0)
    plsc.subcore_barrier()

    pltpu.sync_copy(cnt_sh.at[pl.ds(s * ROWS, ROWS)],
                    cnt_out.at[c, pl.ds(s * ROWS, ROWS)])


def _sc_mesh():
    return plsc.VectorSubcoreMesh(core_axis_name="c", subcore_axis_name="s",
                                  num_cores=NC, num_subcores=NS)


@functools.lru_cache(maxsize=None)
def _make_sc_agg():
    scratch = [
        pltpu.VMEM((2 * 8, K), jnp.int32),
        pltpu.VMEM((2 * 8, K), jnp.int32),
    ] + [pltpu.VMEM((K, D), jnp.float32)] * 4 + [
        pltpu.VMEM_SHARED((NP, D), jnp.float32),
    ] + [pltpu.SemaphoreType.DMA] * 6
    return pl.kernel(_sc_agg_body,
                     out_type=[jax.ShapeDtypeStruct((NC, NP, D), jnp.float32)],
                     mesh=_sc_mesh(), scratch_types=scratch, name="sc_agg")


@functools.lru_cache(maxsize=None)
def _make_sc_count():
    scratch = [
        pltpu.VMEM((2 * 8, K2), jnp.int32),
        pltpu.VMEM((K2, D), jnp.float32),
        pltpu.VMEM_SHARED((NP, D), jnp.float32),
        pltpu.SemaphoreType.DMA,
        pltpu.SemaphoreType.DMA,
    ]
    return pl.kernel(_sc_count_body,
                     out_type=[jax.ShapeDtypeStruct((NC, NP, D), jnp.float32)],
                     mesh=_sc_mesh(), scratch_types=scratch, name="sc_count")


_BN = 2000
_NB = N // _BN


def _tc_layer_common(acc_ref, cnt_ref, h_ref, b_ref, wl_ref, bl_ref, wr_ref,
                     ho_ref, pool_ref):
    i = pl.program_id(0)
    cnt = cnt_ref[0, :, 0:1] + cnt_ref[1, :, 0:1]
    inv = 1.0 / jnp.maximum(cnt, 1.0)
    agg = (acc_ref[0] + acc_ref[1]) * inv
    hmat = h_ref[...]
    out = (lax.dot_general(agg, wl_ref[...], (((1,), (1,)), ((), ())),
                           preferred_element_type=jnp.float32)
           + bl_ref[...]
           + lax.dot_general(hmat, wr_ref[...], (((1,), (1,)), ((), ())),
                             preferred_element_type=jnp.float32))
    ho_ref[...] = out
    onehot_t = (lax.broadcasted_iota(jnp.int32, (G, _BN), 0)
                == b_ref[0]).astype(jnp.float32)
    p = lax.dot_general(onehot_t, out, (((1,), (0,)), ((), ())),
                        preferred_element_type=jnp.float32)

    @pl.when(i == 0)
    def _():
        pool_ref[...] = jnp.zeros_like(pool_ref)

    pool_ref[...] += p


def _tc_layer_body(acc_ref, cnt_ref, h_ref, b_ref, wl_ref, bl_ref, wr_ref,
                   ho_ref, pool_ref):
    _tc_layer_common(acc_ref, cnt_ref, h_ref, b_ref, wl_ref, bl_ref, wr_ref,
                     ho_ref, pool_ref)


def _tc_layer3_body(acc_ref, cnt_ref, h_ref, b_ref, wl_ref, bl_ref, wr_ref,
                    p1_ref, p2_ref, w1_ref, b1_ref, w2_ref, b2_ref,
                    ho_ref, pool_ref, o_ref):
    _tc_layer_common(acc_ref, cnt_ref, h_ref, b_ref, wl_ref, bl_ref, wr_ref,
                     ho_ref, pool_ref)

    @pl.when(pl.program_id(0) == _NB - 1)
    def _():
        dn = (((1,), (1,)), ((), ()))
        z = (lax.dot_general(p1_ref[...], w1_ref[:, 0:H], dn,
                             preferred_element_type=jnp.float32)
             + lax.dot_general(p2_ref[...], w1_ref[:, H:2 * H], dn,
                               preferred_element_type=jnp.float32)
             + lax.dot_general(pool_ref[...], w1_ref[:, 2 * H:3 * H], dn,
                               preferred_element_type=jnp.float32))
        z = jnp.maximum(z + b1_ref[...], 0.0)
        z2 = lax.dot_general(z, w2_ref[...], dn,
                             preferred_element_type=jnp.float32) + b2_ref[...]
        m = jnp.max(z2, axis=-1, keepdims=True)
        lse = jnp.log(jnp.sum(jnp.exp(z2 - m), axis=-1, keepdims=True)) + m
        o_ref[...] = z2 - lse


_tc_layer = pl.pallas_call(
    _tc_layer_body,
    grid=(_NB,),
    in_specs=[
        pl.BlockSpec((NC, _BN, D), lambda i: (0, i, 0)),
        pl.BlockSpec((NC, _BN, D), lambda i: (0, i, 0)),
        pl.BlockSpec((_BN, D), lambda i: (i, 0)),
        pl.BlockSpec((1, 1, _BN), lambda i: (i, 0, 0)),
        pl.BlockSpec((H, D), lambda i: (0, 0)),
        pl.BlockSpec((1, H), lambda i: (0, 0)),
        pl.BlockSpec((H, D), lambda i: (0, 0)),
    ],
    out_specs=[
        pl.BlockSpec((_BN, H), lambda i: (i, 0)),
        pl.BlockSpec((G, H), lambda i: (0, 0)),
    ],
    out_shape=[
        jax.ShapeDtypeStruct((N, H), jnp.float32),
        jax.ShapeDtypeStruct((G, H), jnp.float32),
    ],
)


_tc_layer3 = pl.pallas_call(
    _tc_layer3_body,
    grid=(_NB,),
    in_specs=[
        pl.BlockSpec((NC, _BN, D), lambda i: (0, i, 0)),
        pl.BlockSpec((NC, _BN, D), lambda i: (0, i, 0)),
        pl.BlockSpec((_BN, D), lambda i: (i, 0)),
        pl.BlockSpec((1, 1, _BN), lambda i: (i, 0, 0)),
        pl.BlockSpec((H, D), lambda i: (0, 0)),
        pl.BlockSpec((1, H), lambda i: (0, 0)),
        pl.BlockSpec((H, D), lambda i: (0, 0)),
        pl.BlockSpec((G, H), lambda i: (0, 0)),
        pl.BlockSpec((G, H), lambda i: (0, 0)),
        pl.BlockSpec((H, 3 * H), lambda i: (0, 0)),
        pl.BlockSpec((1, H), lambda i: (0, 0)),
        pl.BlockSpec((C, H), lambda i: (0, 0)),
        pl.BlockSpec((1, C), lambda i: (0, 0)),
    ],
    out_specs=[
        pl.BlockSpec((_BN, H), lambda i: (i, 0)),
        pl.BlockSpec((G, H), lambda i: (0, 0)),
        pl.BlockSpec((G, C), lambda i: (0, 0)),
    ],
    out_shape=[
        jax.ShapeDtypeStruct((N, H), jnp.float32),
        jax.ShapeDtypeStruct((G, H), jnp.float32),
        jax.ShapeDtypeStruct((G, C), jnp.float32),
    ],
)


def _tc_head_body(p1_ref, p2_ref, p3_ref, w1_ref, b1_ref, w2_ref, b2_ref,
                  o_ref):
    dn = (((1,), (1,)), ((), ()))
    z = (lax.dot_general(p1_ref[...], w1_ref[:, 0:H], dn,
                         preferred_element_type=jnp.float32)
         + lax.dot_general(p2_ref[...], w1_ref[:, H:2 * H], dn,
                           preferred_element_type=jnp.float32)
         + lax.dot_general(p3_ref[...], w1_ref[:, 2 * H:3 * H], dn,
                           preferred_element_type=jnp.float32))
    z = jnp.maximum(z + b1_ref[...], 0.0)
    z2 = lax.dot_general(z, w2_ref[...], dn,
                         preferred_element_type=jnp.float32) + b2_ref[...]
    m = jnp.max(z2, axis=-1, keepdims=True)
    lse = jnp.log(jnp.sum(jnp.exp(z2 - m), axis=-1, keepdims=True)) + m
    o_ref[...] = z2 - lse


_tc_head = pl.pallas_call(
    _tc_head_body,
    out_shape=jax.ShapeDtypeStruct((G, C), jnp.float32),
)


def kernel(x, edge_index, batch, Wl0, bl0, Wr0, Wl1, bl1, Wr1, Wl2, bl2,
           Wr2, W1, b1, W2, b2):
    npad = EW - E // NW
    pad_src = (jnp.arange(NW * npad, dtype=jnp.int32) % 64).reshape(NW, npad)
    pad_dst = (N + jnp.arange(NW * npad, dtype=jnp.int32)
               % (NP - N)).reshape(NW, npad)
    src3d = jnp.concatenate(
        [edge_index[0].reshape(NW, E // NW), pad_src], 1).reshape(NW, CH, K)
    dst3d = jnp.concatenate(
        [edge_index[1].reshape(NW, E // NW), pad_dst], 1).reshape(NW, CH, K)
    dst3d2 = dst3d.reshape(NW, CH2, K2)
    zeros_h = jnp.zeros((NP, D), jnp.float32)
    ones_h = jnp.ones((K2, D), jnp.float32)
    batch3d = batch.reshape(_NB, 1, _BN)

    cntp, = _make_sc_count()(dst3d2, zeros_h, ones_h)
    acc, = _make_sc_agg()(x, src3d, dst3d, zeros_h)
    h1, p1 = _tc_layer(acc, cntp, x, batch3d, Wl0, bl0.reshape(1, H), Wr0)
    acc, = _make_sc_agg()(h1, src3d, dst3d, zeros_h)
    h2, p2 = _tc_layer(acc, cntp, h1, batch3d, Wl1, bl1.reshape(1, H), Wr1)
    acc, = _make_sc_agg()(h2, src3d, dst3d, zeros_h)
    _, _, out = _tc_layer3(acc, cntp, h2, batch3d, Wl2, bl2.reshape(1, H),
                           Wr2, p1, p2, W1, b1.reshape(1, H), W2,
                           b2.reshape(1, C))
    return out
